# baseline (speedup 1.0000x reference)
import functools

import jax
import jax.numpy as jnp
from jax import lax
from jax.experimental import pallas as pl
from jax.experimental.pallas import tpu as pltpu
from jax.experimental.pallas import tpu_sc as plsc

_NC = 2
_NS = 16
_NW = _NC * _NS


@functools.cache
def _build_sc_copy(n, d, dtype):
    rows = n // _NW
    mesh = plsc.VectorSubcoreMesh(core_axis_name="c", subcore_axis_name="s")

    @functools.partial(
        pl.kernel,
        out_type=jax.ShapeDtypeStruct((n, d), dtype),
        mesh=mesh,
    )
    def _sc_copy(x_hbm, o_hbm):
        wid = lax.axis_index("s") * _NC + lax.axis_index("c")
        base = wid * rows
        pltpu.sync_copy(x_hbm.at[pl.ds(base, rows)], o_hbm.at[pl.ds(base, rows)])

    return _sc_copy


def kernel(x, W):
    n, d = x.shape
    return _build_sc_copy(n, d, x.dtype)(x)
